# baseline (speedup 1.0000x reference)
import functools
import jax
import jax.numpy as jnp
from jax import lax
from jax.experimental import pallas as pl
from jax.experimental.pallas import tpu as pltpu
from jax.experimental.pallas import tpu_sc as plsc

K = 100000
M = 64
NC = 2
NS = 16
NW = NC * NS
CH = 128
L = 16
SUP = 10
JP = 56


def _make_lookup(J, I):
    assert I == NW * CH and J % SUP == 0
    mesh = plsc.VectorSubcoreMesh(core_axis_name="c", subcore_axis_name="s")

    @functools.partial(
        pl.kernel,
        out_type=jax.ShapeDtypeStruct((J, M, I), jnp.float32),
        mesh=mesh,
        compiler_params=pltpu.CompilerParams(
            use_tc_tiling_on_sc=True, needs_layout_passes=False
        ),
        scratch_types=[
            pltpu.VMEM((JP, CH), jnp.int32),
            pltpu.VMEM((CH,), jnp.int32),
            pltpu.VMEM((CH,), jnp.int32),
            pltpu.VMEM((CH, 2 * M), jnp.float32),
            pltpu.VMEM((CH, 2 * M), jnp.float32),
            pltpu.VMEM((M, CH), jnp.float32),
            pltpu.VMEM((M, CH), jnp.float32),
            pltpu.SemaphoreType.DMA,
            pltpu.SemaphoreType.DMA,
            pltpu.SemaphoreType.DMA,
            pltpu.SemaphoreType.DMA,
        ],
    )
    def lookup(idx_hbm, table2_hbm, out_hbm, idx_v, gi0, gi1, g0, g1, o0, o1,
               gs0, gs1, os0, os1):
        wid = lax.axis_index("s") * NC + lax.axis_index("c")
        col0 = wid * CH
        gix = (gi0, gi1)
        gbuf = (g0, g1)
        obuf = (o0, o1)
        gsem = (gs0, gs1)
        osem = (os0, os1)
        lane = lax.iota(jnp.int32, L)

        pltpu.sync_copy(idx_hbm.at[wid], idx_v)

        def fire_gather(j, b):
            for t in range(CH // L):
                v = idx_v[j, pl.ds(t * L, L)]
                gix[b][pl.ds(t * L, L)] = lax.shift_right_logical(v, 1)
            cp = pltpu.make_async_copy(table2_hbm.at[gix[b]], gbuf[b], gsem[b])
            cp.start()
            return cp

        def out_copy(j, b):
            return pltpu.make_async_copy(
                obuf[b], out_hbm.at[j, :, pl.ds(col0, CH)], osem[b]
            )

        def transpose(j, b):
            cols = []
            rows = []
            for t in range(CH // L):
                v = idx_v[j, pl.ds(t * L, L)]
                cols.append(lax.mul(lax.bitwise_and(v, 1), M))
                rows.append(lane + t * L)

            def body(m, carry):
                for t in range(CH // L):
                    vals = plsc.load_gather(gbuf[b], [rows[t], carry[t]])
                    obuf[b][m, pl.ds(t * L, L)] = vals
                return tuple(c + 1 for c in carry)

            lax.fori_loop(0, M, body, tuple(cols))

        def super_body(j0):
            gcp = [None] * SUP
            ocp = [None] * SUP
            gcp[0] = fire_gather(j0, 0)
            for k in range(SUP):
                b = k % 2
                if k + 1 < SUP:
                    gcp[k + 1] = fire_gather(j0 + k + 1, 1 - b)
                gcp[k].wait()
                transpose(j0 + k, b)
                ocp[k] = out_copy(j0 + k, b)
                ocp[k].start()
                if k >= 1:
                    ocp[k - 1].wait()
            ocp[SUP - 1].wait()

        pl.loop(0, J, step=SUP)(super_body)

    return lookup


def kernel(y, table):
    J = y.shape[1]
    I = y.shape[0]
    table2 = table.reshape(K // 2, 2 * M)
    idx_w = y.T.reshape(J, NW, CH).transpose(1, 0, 2).astype(jnp.int32)
    idx_w = jnp.pad(idx_w, ((0, 0), (0, JP - J), (0, 0)))
    out_t = _make_lookup(J, I)(idx_w, table2)
    return out_t.transpose(2, 0, 1)


# speedup vs baseline: 1.4786x; 1.4786x over previous
import functools
import jax
import jax.numpy as jnp
from jax import lax
from jax.experimental import pallas as pl
from jax.experimental.pallas import tpu as pltpu
from jax.experimental.pallas import tpu_sc as plsc

K = 100000
M = 64
NC = 2
NS = 16
NW = NC * NS
L = 16
SUP = 10
MPW = M // NW


def _make_lookup(J, I):
    assert J % SUP == 0
    mesh = plsc.VectorSubcoreMesh(core_axis_name="c", subcore_axis_name="s")

    @functools.partial(
        pl.kernel,
        out_type=jax.ShapeDtypeStruct((J, M, I), jnp.float32),
        mesh=mesh,
        compiler_params=pltpu.CompilerParams(
            use_tc_tiling_on_sc=True, needs_layout_passes=False
        ),
        scratch_types=[
            pltpu.VMEM((K,), jnp.float32),
            pltpu.VMEM((I,), jnp.int32),
            pltpu.VMEM((I,), jnp.int32),
            pltpu.VMEM((I,), jnp.float32),
            pltpu.VMEM((I,), jnp.float32),
            pltpu.SemaphoreType.DMA,
            pltpu.SemaphoreType.DMA,
            pltpu.SemaphoreType.DMA,
            pltpu.SemaphoreType.DMA,
        ],
    )
    def lookup(yflat_hbm, tt_hbm, out_hbm, row_v, ix0, ix1, ob0, ob1,
               is0, is1, os0, os1):
        wid = lax.axis_index("s") * NC + lax.axis_index("c")
        ixb = (ix0, ix1)
        obuf = (ob0, ob1)
        isem = (is0, is1)
        osem = (os0, os1)

        def idx_load(j, b):
            cp = pltpu.make_async_copy(
                yflat_hbm.at[pl.ds(j * I, I)], ixb[b], isem[b]
            )
            cp.start()
            return cp

        def gather_chunk(b):
            @pl.loop(0, I // L, unroll=8)
            def _(q):
                iv = ixb[b][pl.ds(q * L, L)]
                obuf[b][pl.ds(q * L, L)] = plsc.load_gather(row_v, [iv])

        def out_copy(j, m, b):
            return pltpu.make_async_copy(obuf[b], out_hbm.at[j, m], osem[b])

        for mi in range(MPW):
            m = wid * MPW + mi
            pltpu.sync_copy(tt_hbm.at[m], row_v)

            def super_body(j0):
                icp = [None] * SUP
                ocp = [None] * SUP
                icp[0] = idx_load(j0, 0)
                for k in range(SUP):
                    b = k % 2
                    if k + 1 < SUP:
                        icp[k + 1] = idx_load(j0 + k + 1, 1 - b)
                    icp[k].wait()
                    gather_chunk(b)
                    ocp[k] = out_copy(j0 + k, m, b)
                    ocp[k].start()
                    if k >= 1:
                        ocp[k - 1].wait()
                ocp[SUP - 1].wait()

            pl.loop(0, J, step=SUP)(super_body)

    return lookup


def kernel(y, table):
    I, J = y.shape
    y_flat = y.T.reshape(I * J).astype(jnp.int32)
    table_t = table.T
    out_t = _make_lookup(J, I)(y_flat, table_t)
    return out_t.transpose(2, 0, 1)


# speedup vs baseline: 3.3030x; 2.2339x over previous
import functools
import jax
import jax.numpy as jnp
from jax import lax
from jax.experimental import pallas as pl
from jax.experimental.pallas import tpu as pltpu
from jax.experimental.pallas import tpu_sc as plsc

K = 100000
M = 64
NC = 2
NS = 16
NW = NC * NS
L = 16
SUP = 10
MPW = M // NW


def _make_lookup(J, I):
    assert J % SUP == 0
    mesh = plsc.VectorSubcoreMesh(core_axis_name="c", subcore_axis_name="s")

    @functools.partial(
        pl.kernel,
        out_type=jax.ShapeDtypeStruct((J, M, I), jnp.float32),
        mesh=mesh,
        compiler_params=pltpu.CompilerParams(
            use_tc_tiling_on_sc=True, needs_layout_passes=False
        ),
        scratch_types=[
            pltpu.VMEM((K,), jnp.float32),
            pltpu.VMEM((I,), jnp.int32),
            pltpu.VMEM((I,), jnp.int32),
            pltpu.VMEM((I,), jnp.float32),
            pltpu.VMEM((I,), jnp.float32),
            pltpu.SemaphoreType.DMA,
            pltpu.SemaphoreType.DMA,
            pltpu.SemaphoreType.DMA,
            pltpu.SemaphoreType.DMA,
        ],
    )
    def lookup(yflat_hbm, tt_hbm, out_hbm, row_v, ix0, ix1, ob0, ob1,
               is0, is1, os0, os1):
        wid = lax.axis_index("s") * NC + lax.axis_index("c")
        ixb = (ix0, ix1)
        obuf = (ob0, ob1)
        isem = (is0, is1)
        osem = (os0, os1)

        def idx_load(j, b):
            cp = pltpu.make_async_copy(
                yflat_hbm.at[pl.ds(j * I, I)], ixb[b], isem[b]
            )
            cp.start()
            return cp

        def gather_chunk(b):
            @plsc.parallel_loop(0, I // L, unroll=8)
            def _(q):
                iv = ixb[b][pl.ds(q * L, L)]
                obuf[b][pl.ds(q * L, L)] = plsc.load_gather(row_v, [iv])

        def out_copy(j, m, b):
            return pltpu.make_async_copy(obuf[b], out_hbm.at[j, m], osem[b])

        for mi in range(MPW):
            m = wid * MPW + mi
            pltpu.sync_copy(tt_hbm.at[m], row_v)

            def super_body(j0):
                icp = [None] * SUP
                ocp = [None] * SUP
                icp[0] = idx_load(j0, 0)
                for k in range(SUP):
                    b = k % 2
                    if k + 1 < SUP:
                        icp[k + 1] = idx_load(j0 + k + 1, 1 - b)
                    icp[k].wait()
                    gather_chunk(b)
                    ocp[k] = out_copy(j0 + k, m, b)
                    ocp[k].start()
                    if k >= 1:
                        ocp[k - 1].wait()
                ocp[SUP - 1].wait()

            pl.loop(0, J, step=SUP)(super_body)

    return lookup


def kernel(y, table):
    I, J = y.shape
    y_flat = y.T.reshape(I * J).astype(jnp.int32)
    table_t = table.T
    out_t = _make_lookup(J, I)(y_flat, table_t)
    return out_t.transpose(2, 0, 1)
